# baseline (speedup 1.0000x reference)
import functools

import jax
import jax.numpy as jnp
from jax import lax
from jax.experimental import pallas as pl
from jax.experimental.pallas import tpu as pltpu
from jax.experimental.pallas import tpu_sc as plsc

_C = 1000
_B = 4096
_D = 2048
_NW = 32
_DC = 32
_NP = 2
_R = 256
_TR = _R // 8
_NCH = _B // _R
_CK = 200
_NK = _C // _CK
_L = 16


def _body(x_hbm, lab_hbm, rm_hbm, rv_hbm, cc_hbm,
          nm_hbm, nv_hbm, ncnt_hbm, preg_hbm,
          labv, xab, sumt, sum2t, rmc, rvc,
          cci, cnti, ncv, af, rf, nfv, t1r, t2r, t3r, pregv,
          sx, sm0, sm1, sv0, sv1):
    w = lax.axis_index("s") * 2 + lax.axis_index("c")

    cols = [(w + p * _NW) * _DC for p in range(_NP)]
    g0s = [lax.shift_right_logical(c, 7) for c in cols]
    o0s = [pl.multiple_of(lax.bitwise_and(c, 127), _DC) for c in cols]

    pltpu.async_copy(
        x_hbm.at[pl.ds(0, _TR), g0s[0], :, pl.ds(o0s[0], _DC)],
        xab.at[0], sx.at[0])

    pltpu.sync_copy(lab_hbm, labv.at[pl.ds(0, _B)])

    def zc(g, _):
        z = jnp.zeros((_L,), jnp.int32)
        cci[pl.ds(g * _L, _L)] = z
        cnti[pl.ds(g * _L, _L)] = z
        return _
    lax.fori_loop(0, 1024 // _L, zc, None, unroll=8)
    pltpu.sync_copy(cc_hbm, cci.at[pl.ds(0, _C)])
    t1r[...] = jnp.zeros((_L,), jnp.float32)
    t2r[...] = jnp.zeros((_L,), jnp.float32)
    t3r[...] = jnp.zeros((_L,), jnp.float32)

    ones_i = jnp.ones((_L,), jnp.int32)

    def cbody(g, _):
        idx = labv[pl.ds(g * _L, _L)]
        plsc.addupdate_scatter(cnti, [idx], ones_i)
        return _
    lax.fori_loop(0, _B // _L, cbody, None, unroll=8)

    def pbody(g, _):
        s = pl.ds(g * _L, _L)
        ci = cnti[s]
        cc = cci[s]
        ccn = ci + cc
        ncv[s] = ccn
        pos = ccn > 0
        den = jnp.where(pos, ccn.astype(jnp.float32), 1.0)
        r = 1.0 / den
        af[s] = jnp.where(pos, cc.astype(jnp.float32) * r, 1.0)
        rf[s] = r
        nfv[s] = ci.astype(jnp.float32)
        return _
    lax.fori_loop(0, 1024 // _L, pbody, None, unroll=4)

    @pl.when(w == 0)
    def _():
        pltpu.sync_copy(ncv.at[pl.ds(0, _C)], ncnt_hbm)

    msems = (sm0, sm1)
    vsems = (sv0, sv1)
    ho = []

    for p in range(_NP):
        g0, o0 = g0s[p], o0s[p]

        hm = [None] * _NK
        hv = [None] * _NK
        hm[0] = pltpu.async_copy(
            rm_hbm.at[pl.ds(0, _CK // 8), g0, :, pl.ds(o0, _DC)],
            rmc.at[0], sm0)
        hv[0] = pltpu.async_copy(
            rv_hbm.at[pl.ds(0, _CK // 8), g0, :, pl.ds(o0, _DC)],
            rvc.at[0], sv0)

        for hprev in ho:
            hprev.wait()
        ho = []

        def zb(i, _):
            z = jnp.zeros((_L,), jnp.float32)
            for s8 in range(8):
                sumt[i, s8, pl.ds(0, _L)] = z
                sumt[i, s8, pl.ds(_L, _L)] = z
                sum2t[i, s8, pl.ds(0, _L)] = z
                sum2t[i, s8, pl.ds(_L, _L)] = z
            return _
        lax.fori_loop(0, _C // 8, zb, None, unroll=2)

        def chunk_body(ch, _):
            nxt = lax.rem(ch + 1, 2)
            cur = lax.rem(ch, 2)

            @pl.when(ch + 1 < _NCH)
            def _():
                pltpu.async_copy(
                    x_hbm.at[pl.ds((ch + 1) * _TR, _TR), g0, :,
                             pl.ds(o0, _DC)],
                    xab.at[nxt], sx.at[nxt])
            pltpu.make_async_copy(
                x_hbm.at[pl.ds(ch * _TR, _TR), g0, :, pl.ds(o0, _DC)],
                xab.at[cur], sx.at[cur]).wait()

            def rbody(tr, _):
                lv = labv[pl.ds(ch * _R + tr * 8, _L)]
                for s8 in range(8):
                    l = lv[s8]
                    l8 = lax.shift_right_logical(l, 3)
                    lr = lax.bitwise_and(l, 7)
                    v0 = xab[cur, tr, s8, pl.ds(0, _L)]
                    v1 = xab[cur, tr, s8, pl.ds(_L, _L)]
                    plsc.addupdate(sumt.at[l8, lr, pl.ds(0, _L)], v0)
                    plsc.addupdate(sumt.at[l8, lr, pl.ds(_L, _L)], v1)
                    plsc.addupdate(sum2t.at[l8, lr, pl.ds(0, _L)], v0 * v0)
                    plsc.addupdate(sum2t.at[l8, lr, pl.ds(_L, _L)], v1 * v1)
                return _
            lax.fori_loop(0, _TR, rbody, None, unroll=4)
            return _
        lax.fori_loop(0, _NCH, chunk_body, None)
        if p + 1 < _NP:
            pltpu.async_copy(
                x_hbm.at[pl.ds(0, _TR), g0s[p + 1], :,
                         pl.ds(o0s[p + 1], _DC)], xab.at[0], sx.at[0])

        for k in range(_NK):
            if k + 1 < _NK:
                hm[k + 1] = pltpu.async_copy(
                    rm_hbm.at[pl.ds((k + 1) * _CK // 8, _CK // 8), g0, :,
                              pl.ds(o0, _DC)],
                    rmc.at[(k + 1) % 2], msems[(k + 1) % 2])
                hv[k + 1] = pltpu.async_copy(
                    rv_hbm.at[pl.ds((k + 1) * _CK // 8, _CK // 8), g0, :,
                              pl.ds(o0, _DC)],
                    rvc.at[(k + 1) % 2], vsems[(k + 1) % 2])
            hm[k].wait()
            hv[k].wait()
            kb = k % 2

            def dbody(c, _):
                cls = k * _CK + c
                c8 = lax.shift_right_logical(c, 3)
                cr = lax.bitwise_and(c, 7)
                t8 = lax.shift_right_logical(cls, 3)
                tr_ = lax.bitwise_and(cls, 7)
                a = jnp.full((_L,), af[pl.ds(cls, _L)][0])
                r = jnp.full((_L,), rf[pl.ds(cls, _L)][0])
                nn = jnp.full((_L,), nfv[pl.ds(cls, _L)][0])
                m0 = rmc[kb, c8, cr, pl.ds(0, _L)]
                m1 = rmc[kb, c8, cr, pl.ds(_L, _L)]
                v0 = rvc[kb, c8, cr, pl.ds(0, _L)]
                v1 = rvc[kb, c8, cr, pl.ds(_L, _L)]
                s0 = sumt[t8, tr_, pl.ds(0, _L)]
                s1 = sumt[t8, tr_, pl.ds(_L, _L)]
                q0 = sum2t[t8, tr_, pl.ds(0, _L)]
                q1 = sum2t[t8, tr_, pl.ds(_L, _L)]
                plsc.addupdate(t1r.at[pl.ds(0, _L)], q0 + q1)
                plsc.addupdate(t2r.at[pl.ds(0, _L)], s0 * m0 + s1 * m1)
                plsc.addupdate(t3r.at[pl.ds(0, _L)], nn * (m0 * m0 + m1 * m1))
                sumt[t8, tr_, pl.ds(0, _L)] = m0 * a + s0 * r
                sumt[t8, tr_, pl.ds(_L, _L)] = m1 * a + s1 * r
                sum2t[t8, tr_, pl.ds(0, _L)] = v0 * a + q0 * r
                sum2t[t8, tr_, pl.ds(_L, _L)] = v1 * a + q1 * r
                return _
            lax.fori_loop(0, _CK, dbody, None, unroll=4)

        ho = [
            pltpu.async_copy(sumt, nm_hbm.at[:, g0, :, pl.ds(o0, _DC)], sm0),
            pltpu.async_copy(sum2t, nv_hbm.at[:, g0, :, pl.ds(o0, _DC)], sv0),
        ]

    for hprev in ho:
        hprev.wait()
    pregv[...] = t1r[...] - 2.0 * t2r[...] + t3r[...]
    pltpu.sync_copy(pregv, preg_hbm.at[pl.ds(w * _L, _L)])


def kernel(x, labels, running_mean, running_var, class_count):
    f32 = jnp.float32
    x4 = x.reshape(_B // 8, 8, _D // 128, 128).transpose(0, 2, 1, 3)
    rm4 = running_mean.reshape(_C // 8, 8, _D // 128, 128).transpose(0, 2, 1, 3)
    rv4 = running_var.reshape(_C // 8, 8, _D // 128, 128).transpose(0, 2, 1, 3)
    cc1 = class_count.reshape(_C)
    mesh = plsc.VectorSubcoreMesh(core_axis_name="c", subcore_axis_name="s")
    run = functools.partial(
        pl.kernel,
        mesh=mesh,
        compiler_params=pltpu.CompilerParams(
            use_tc_tiling_on_sc=False, needs_layout_passes=False),
        out_type=(
            jax.ShapeDtypeStruct((_C // 8, _D // 128, 8, 128), f32),
            jax.ShapeDtypeStruct((_C // 8, _D // 128, 8, 128), f32),
            jax.ShapeDtypeStruct((_C,), jnp.int32),
            jax.ShapeDtypeStruct((_NW * _L,), f32),
        ),
        scratch_types=[
            pltpu.VMEM((_B + _L,), jnp.int32),
            pltpu.VMEM((2, _TR, 8, _DC), f32),
            pltpu.VMEM((_C // 8, 8, _DC), f32),
            pltpu.VMEM((_C // 8, 8, _DC), f32),
            pltpu.VMEM((2, _CK // 8, 8, _DC), f32),
            pltpu.VMEM((2, _CK // 8, 8, _DC), f32),
            pltpu.VMEM((1024,), jnp.int32),
            pltpu.VMEM((1024,), jnp.int32),
            pltpu.VMEM((1024,), jnp.int32),
            pltpu.VMEM((1024,), f32),
            pltpu.VMEM((1024,), f32),
            pltpu.VMEM((1024,), f32),
            pltpu.VMEM((_L,), f32),
            pltpu.VMEM((_L,), f32),
            pltpu.VMEM((_L,), f32),
            pltpu.VMEM((_L,), f32),
            pltpu.SemaphoreType.DMA((2,)),
            pltpu.SemaphoreType.DMA,
            pltpu.SemaphoreType.DMA,
            pltpu.SemaphoreType.DMA,
            pltpu.SemaphoreType.DMA,
        ],
    )(_body)
    nm4, nv4, nc, pr = run(x4, labels, rm4, rv4, cc1)
    nm = nm4.transpose(0, 2, 1, 3).reshape(_C, _D)
    nv = nv4.transpose(0, 2, 1, 3).reshape(_C, _D)
    return nm, nv, nc.reshape(_C, 1), jnp.sqrt(jnp.sum(pr))


# speedup vs baseline: 1.0159x; 1.0159x over previous
import functools

import jax
import jax.numpy as jnp
from jax import lax
from jax.experimental import pallas as pl
from jax.experimental.pallas import tpu as pltpu
from jax.experimental.pallas import tpu_sc as plsc

_C = 1000
_B = 4096
_D = 2048
_NW = 32
_DC = 32
_NP = 2
_R = 256
_TR = _R // 8
_NCH = _B // _R
_CK = 200
_NK = _C // _CK
_L = 16


def _body(x_hbm, lab_hbm, rm_hbm, rv_hbm, cc_hbm,
          nm_hbm, nv_hbm, ncnt_hbm, preg_hbm,
          labv, xab, sumt, sum2t, rmc, rvc,
          cci, cnti, ncv, af, rf, nfv, t1r, t2r, t3r, pregv,
          sx, sm0, sm1, sv0, sv1):
    w = lax.axis_index("s") * 2 + lax.axis_index("c")

    cols = [(w + p * _NW) * _DC for p in range(_NP)]
    g0s = [lax.shift_right_logical(c, 7) for c in cols]
    o0s = [pl.multiple_of(lax.bitwise_and(c, 127), _DC) for c in cols]

    pltpu.async_copy(
        x_hbm.at[pl.ds(0, _TR), g0s[0], :, pl.ds(o0s[0], _DC)],
        xab.at[0], sx.at[0])

    hlab = pltpu.async_copy(lab_hbm, labv.at[pl.ds(0, _B)], sm1)

    def zc(g, _):
        z = jnp.zeros((_L,), jnp.int32)
        cci[pl.ds(g * _L, _L)] = z
        cnti[pl.ds(g * _L, _L)] = z
        return _
    lax.fori_loop(0, 1024 // _L, zc, None, unroll=8)
    pltpu.sync_copy(cc_hbm, cci.at[pl.ds(0, _C)])
    hlab.wait()
    t1r[...] = jnp.zeros((_L,), jnp.float32)
    t2r[...] = jnp.zeros((_L,), jnp.float32)
    t3r[...] = jnp.zeros((_L,), jnp.float32)

    ones_i = jnp.ones((_L,), jnp.int32)

    def cbody(g, _):
        idx = labv[pl.ds(g * _L, _L)]
        plsc.addupdate_scatter(cnti, [idx], ones_i)
        return _
    lax.fori_loop(0, _B // _L, cbody, None, unroll=8)

    def pbody(g, _):
        s = pl.ds(g * _L, _L)
        ci = cnti[s]
        cc = cci[s]
        ccn = ci + cc
        ncv[s] = ccn
        pos = ccn > 0
        den = jnp.where(pos, ccn.astype(jnp.float32), 1.0)
        r = 1.0 / den
        af[s] = jnp.where(pos, cc.astype(jnp.float32) * r, 1.0)
        rf[s] = r
        nfv[s] = ci.astype(jnp.float32)
        return _
    lax.fori_loop(0, 1024 // _L, pbody, None, unroll=4)

    @pl.when(w == 0)
    def _():
        pltpu.sync_copy(ncv.at[pl.ds(0, _C)], ncnt_hbm)

    msems = (sm0, sm1)
    vsems = (sv0, sv1)
    ho = []

    for p in range(_NP):
        g0, o0 = g0s[p], o0s[p]

        hm = [None] * _NK
        hv = [None] * _NK
        hm[0] = pltpu.async_copy(
            rm_hbm.at[pl.ds(0, _CK // 8), g0, :, pl.ds(o0, _DC)],
            rmc.at[0], sm0)
        hv[0] = pltpu.async_copy(
            rv_hbm.at[pl.ds(0, _CK // 8), g0, :, pl.ds(o0, _DC)],
            rvc.at[0], sv0)

        for hprev in ho:
            hprev.wait()
        ho = []

        def zb(i, _):
            z = jnp.zeros((_L,), jnp.float32)
            for s8 in range(8):
                sumt[i, s8, pl.ds(0, _L)] = z
                sumt[i, s8, pl.ds(_L, _L)] = z
                sum2t[i, s8, pl.ds(0, _L)] = z
                sum2t[i, s8, pl.ds(_L, _L)] = z
            return _
        lax.fori_loop(0, _C // 8, zb, None, unroll=2)

        def chunk_body(ch, _):
            nxt = lax.rem(ch + 1, 2)
            cur = lax.rem(ch, 2)

            @pl.when(ch + 1 < _NCH)
            def _():
                pltpu.async_copy(
                    x_hbm.at[pl.ds((ch + 1) * _TR, _TR), g0, :,
                             pl.ds(o0, _DC)],
                    xab.at[nxt], sx.at[nxt])
            pltpu.make_async_copy(
                x_hbm.at[pl.ds(ch * _TR, _TR), g0, :, pl.ds(o0, _DC)],
                xab.at[cur], sx.at[cur]).wait()

            def rbody(tr, _):
                lv = labv[pl.ds(ch * _R + tr * 8, _L)]
                for s8 in range(8):
                    l = lv[s8]
                    l8 = lax.shift_right_logical(l, 3)
                    lr = lax.bitwise_and(l, 7)
                    v0 = xab[cur, tr, s8, pl.ds(0, _L)]
                    v1 = xab[cur, tr, s8, pl.ds(_L, _L)]
                    plsc.addupdate(sumt.at[l8, lr, pl.ds(0, _L)], v0)
                    plsc.addupdate(sumt.at[l8, lr, pl.ds(_L, _L)], v1)
                    plsc.addupdate(sum2t.at[l8, lr, pl.ds(0, _L)], v0 * v0)
                    plsc.addupdate(sum2t.at[l8, lr, pl.ds(_L, _L)], v1 * v1)
                return _
            lax.fori_loop(0, _TR, rbody, None, unroll=2)
            return _
        lax.fori_loop(0, _NCH, chunk_body, None)
        if p + 1 < _NP:
            pltpu.async_copy(
                x_hbm.at[pl.ds(0, _TR), g0s[p + 1], :,
                         pl.ds(o0s[p + 1], _DC)], xab.at[0], sx.at[0])

        for k in range(_NK):
            if k + 1 < _NK:
                hm[k + 1] = pltpu.async_copy(
                    rm_hbm.at[pl.ds((k + 1) * _CK // 8, _CK // 8), g0, :,
                              pl.ds(o0, _DC)],
                    rmc.at[(k + 1) % 2], msems[(k + 1) % 2])
                hv[k + 1] = pltpu.async_copy(
                    rv_hbm.at[pl.ds((k + 1) * _CK // 8, _CK // 8), g0, :,
                              pl.ds(o0, _DC)],
                    rvc.at[(k + 1) % 2], vsems[(k + 1) % 2])
            hm[k].wait()
            hv[k].wait()
            kb = k % 2

            def dbody(c, _):
                cls = k * _CK + c
                c8 = lax.shift_right_logical(c, 3)
                cr = lax.bitwise_and(c, 7)
                t8 = lax.shift_right_logical(cls, 3)
                tr_ = lax.bitwise_and(cls, 7)
                a = jnp.full((_L,), af[pl.ds(cls, _L)][0])
                r = jnp.full((_L,), rf[pl.ds(cls, _L)][0])
                nn = jnp.full((_L,), nfv[pl.ds(cls, _L)][0])
                m0 = rmc[kb, c8, cr, pl.ds(0, _L)]
                m1 = rmc[kb, c8, cr, pl.ds(_L, _L)]
                v0 = rvc[kb, c8, cr, pl.ds(0, _L)]
                v1 = rvc[kb, c8, cr, pl.ds(_L, _L)]
                s0 = sumt[t8, tr_, pl.ds(0, _L)]
                s1 = sumt[t8, tr_, pl.ds(_L, _L)]
                q0 = sum2t[t8, tr_, pl.ds(0, _L)]
                q1 = sum2t[t8, tr_, pl.ds(_L, _L)]
                plsc.addupdate(t1r.at[pl.ds(0, _L)], q0 + q1)
                plsc.addupdate(t2r.at[pl.ds(0, _L)], s0 * m0 + s1 * m1)
                plsc.addupdate(t3r.at[pl.ds(0, _L)], nn * (m0 * m0 + m1 * m1))
                sumt[t8, tr_, pl.ds(0, _L)] = m0 * a + s0 * r
                sumt[t8, tr_, pl.ds(_L, _L)] = m1 * a + s1 * r
                sum2t[t8, tr_, pl.ds(0, _L)] = v0 * a + q0 * r
                sum2t[t8, tr_, pl.ds(_L, _L)] = v1 * a + q1 * r
                return _
            lax.fori_loop(0, _CK, dbody, None, unroll=4)

        ho = [
            pltpu.async_copy(sumt, nm_hbm.at[:, g0, :, pl.ds(o0, _DC)], sm0),
            pltpu.async_copy(sum2t, nv_hbm.at[:, g0, :, pl.ds(o0, _DC)], sv0),
        ]

    for hprev in ho:
        hprev.wait()
    pregv[...] = t1r[...] - 2.0 * t2r[...] + t3r[...]
    pltpu.sync_copy(pregv, preg_hbm.at[pl.ds(w * _L, _L)])


def kernel(x, labels, running_mean, running_var, class_count):
    f32 = jnp.float32
    x4 = x.reshape(_B // 8, 8, _D // 128, 128).transpose(0, 2, 1, 3)
    rm4 = running_mean.reshape(_C // 8, 8, _D // 128, 128).transpose(0, 2, 1, 3)
    rv4 = running_var.reshape(_C // 8, 8, _D // 128, 128).transpose(0, 2, 1, 3)
    cc1 = class_count.reshape(_C)
    mesh = plsc.VectorSubcoreMesh(core_axis_name="c", subcore_axis_name="s")
    run = functools.partial(
        pl.kernel,
        mesh=mesh,
        compiler_params=pltpu.CompilerParams(
            use_tc_tiling_on_sc=False, needs_layout_passes=False),
        out_type=(
            jax.ShapeDtypeStruct((_C // 8, _D // 128, 8, 128), f32),
            jax.ShapeDtypeStruct((_C // 8, _D // 128, 8, 128), f32),
            jax.ShapeDtypeStruct((_C,), jnp.int32),
            jax.ShapeDtypeStruct((_NW * _L,), f32),
        ),
        scratch_types=[
            pltpu.VMEM((_B + _L,), jnp.int32),
            pltpu.VMEM((2, _TR, 8, _DC), f32),
            pltpu.VMEM((_C // 8, 8, _DC), f32),
            pltpu.VMEM((_C // 8, 8, _DC), f32),
            pltpu.VMEM((2, _CK // 8, 8, _DC), f32),
            pltpu.VMEM((2, _CK // 8, 8, _DC), f32),
            pltpu.VMEM((1024,), jnp.int32),
            pltpu.VMEM((1024,), jnp.int32),
            pltpu.VMEM((1024,), jnp.int32),
            pltpu.VMEM((1024,), f32),
            pltpu.VMEM((1024,), f32),
            pltpu.VMEM((1024,), f32),
            pltpu.VMEM((_L,), f32),
            pltpu.VMEM((_L,), f32),
            pltpu.VMEM((_L,), f32),
            pltpu.VMEM((_L,), f32),
            pltpu.SemaphoreType.DMA((2,)),
            pltpu.SemaphoreType.DMA,
            pltpu.SemaphoreType.DMA,
            pltpu.SemaphoreType.DMA,
            pltpu.SemaphoreType.DMA,
        ],
    )(_body)
    nm4, nv4, nc, pr = run(x4, labels, rm4, rv4, cc1)
    nm = nm4.transpose(0, 2, 1, 3).reshape(_C, _D)
    nv = nv4.transpose(0, 2, 1, 3).reshape(_C, _D)
    return nm, nv, nc.reshape(_C, 1), jnp.sqrt(jnp.sum(pr))
